# baseline (speedup 1.0000x reference)
import jax
import jax.numpy as jnp
from jax.experimental import pallas as pl
from jax.experimental.pallas import tpu as pltpu

T = 2048
D = 1024
E = 8
FF = 2048
FJ = 8
FB = FF // FJ


def _moe_body(hr, hbr, rwr, rbr, gwr, uwr, dwr, sgr, sur, sdr,
              out_ref, comb_ref):
    e = pl.program_id(0)
    j = pl.program_id(1)

    @pl.when((e == 0) & (j == 0))
    def _init():
        h = hr[...]
        scores = jax.nn.sigmoid(
            jax.lax.dot_general(h, rwr[...], (((1,), (1,)), ((), ())),
                                preferred_element_type=jnp.float32))
        s = scores + rbr[...]
        lane = jax.lax.broadcasted_iota(jnp.int32, (T, E), 1)
        m1 = jnp.max(s, axis=1, keepdims=True)
        i1 = jnp.min(jnp.where(s == m1, lane, E), axis=1, keepdims=True)
        mask1 = lane == i1
        s2 = jnp.where(mask1, -jnp.inf, s)
        m2 = jnp.max(s2, axis=1, keepdims=True)
        i2 = jnp.min(jnp.where(s2 == m2, lane, E), axis=1, keepdims=True)
        mask2 = lane == i2
        w1 = jnp.sum(jnp.where(mask1, scores, 0.0), axis=1, keepdims=True)
        w2 = jnp.sum(jnp.where(mask2, scores, 0.0), axis=1, keepdims=True)
        denom = w1 + w2 + 1e-20
        comb_ref[...] = (jnp.where(mask1, w1, 0.0)
                         + jnp.where(mask2, w2, 0.0)) / denom
        out_ref[...] = jnp.zeros_like(out_ref)

    is_sh = e == E
    gw = jnp.where(is_sh, sgr[...], gwr[0])
    uw = jnp.where(is_sh, sur[...], uwr[0])
    dw = jnp.where(is_sh, sdr[...], dwr[0])

    lane = jax.lax.broadcasted_iota(jnp.int32, (T, E), 1)
    wcol = jnp.sum(jnp.where(lane == e, comb_ref[...], 0.0),
                   axis=1, keepdims=True)
    wcol = jnp.where(is_sh, 1.0, wcol)

    hb = hbr[...]
    g = jax.lax.dot_general(hb, gw, (((1,), (1,)), ((), ())),
                            preferred_element_type=jnp.float32)
    u = jax.lax.dot_general(hb, uw, (((1,), (1,)), ((), ())),
                            preferred_element_type=jnp.float32)
    a = ((g * jax.nn.sigmoid(g)) * u).astype(jnp.bfloat16)
    p = jax.lax.dot_general(a, dw, (((1,), (1,)), ((), ())),
                            preferred_element_type=jnp.float32)
    out_ref[...] += wcol * p


def kernel(x, router_w, router_bias, gate_w, up_w, down_w,
           sh_gate_w, sh_up_w, sh_down_w):
    h = x.reshape(T, D)
    hb = h.astype(jnp.bfloat16)
    rb = router_bias.reshape(1, E)
    gwb = gate_w.astype(jnp.bfloat16)
    uwb = up_w.astype(jnp.bfloat16)
    dwb = down_w.astype(jnp.bfloat16)
    sgb = sh_gate_w.astype(jnp.bfloat16)
    sub = sh_up_w.astype(jnp.bfloat16)
    sdb = sh_down_w.astype(jnp.bfloat16)
    out = pl.pallas_call(
        _moe_body,
        grid=(E + 1, FJ),
        in_specs=[
            pl.BlockSpec((T, D), lambda e, j: (0, 0)),
            pl.BlockSpec((T, D), lambda e, j: (0, 0)),
            pl.BlockSpec((E, D), lambda e, j: (0, 0)),
            pl.BlockSpec((1, E), lambda e, j: (0, 0)),
            pl.BlockSpec((1, FB, D),
                         lambda e, j: (jnp.minimum(e, E - 1), j, 0)),
            pl.BlockSpec((1, FB, D),
                         lambda e, j: (jnp.minimum(e, E - 1), j, 0)),
            pl.BlockSpec((1, D, FB),
                         lambda e, j: (jnp.minimum(e, E - 1), 0, j)),
            pl.BlockSpec((FB, D), lambda e, j: (jnp.where(e == E, j, 0), 0)),
            pl.BlockSpec((FB, D), lambda e, j: (jnp.where(e == E, j, 0), 0)),
            pl.BlockSpec((D, FB), lambda e, j: (0, jnp.where(e == E, j, 0))),
        ],
        out_specs=pl.BlockSpec((T, D), lambda e, j: (0, 0)),
        out_shape=jax.ShapeDtypeStruct((T, D), jnp.float32),
        scratch_shapes=[pltpu.VMEM((T, E), jnp.float32)],
        compiler_params=pltpu.CompilerParams(
            dimension_semantics=("arbitrary", "arbitrary")),
    )(h, hb, router_w, rb, gwb, uwb, dwb, sgb, sub, sdb)
    return out.reshape(x.shape)


# speedup vs baseline: 1.2962x; 1.2962x over previous
import jax
import jax.numpy as jnp
from jax.experimental import pallas as pl
from jax.experimental.pallas import tpu as pltpu

T = 2048
D = 1024
E = 8
FF = 2048
FJ = 4
FB = FF // FJ
SB = FF // (2 * FJ)


def _moe_body(hr, rwr, rbr, gwr, uwr, dwr, sgr, sur, sdr, out_ref, comb_ref):
    e = pl.program_id(0)
    j = pl.program_id(1)

    h = hr[...]

    @pl.when((e == 0) & (j == 0))
    def _init():
        scores = jax.nn.sigmoid(
            jax.lax.dot_general(h, rwr[...], (((1,), (1,)), ((), ())),
                                preferred_element_type=jnp.float32))
        s = scores + rbr[...]
        lane = jax.lax.broadcasted_iota(jnp.int32, (T, E), 1)
        m1 = jnp.max(s, axis=1, keepdims=True)
        i1 = jnp.min(jnp.where(s == m1, lane, E), axis=1, keepdims=True)
        mask1 = lane == i1
        s2 = jnp.where(mask1, -jnp.inf, s)
        m2 = jnp.max(s2, axis=1, keepdims=True)
        i2 = jnp.min(jnp.where(s2 == m2, lane, E), axis=1, keepdims=True)
        mask2 = lane == i2
        w1 = jnp.sum(jnp.where(mask1, scores, 0.0), axis=1, keepdims=True)
        w2 = jnp.sum(jnp.where(mask2, scores, 0.0), axis=1, keepdims=True)
        denom = w1 + w2 + 1e-20
        comb_ref[...] = (jnp.where(mask1, w1, 0.0)
                         + jnp.where(mask2, w2, 0.0)) / denom
        out_ref[...] = jnp.zeros_like(out_ref)

    @pl.when(e < E)
    def _routed():
        lane = jax.lax.broadcasted_iota(jnp.int32, (T, E), 1)
        wcol = jnp.sum(jnp.where(lane == e, comb_ref[...], 0.0),
                       axis=1, keepdims=True)
        g = jax.lax.dot_general(h, gwr[0], (((1,), (1,)), ((), ())),
                                preferred_element_type=jnp.float32)
        u = jax.lax.dot_general(h, uwr[0], (((1,), (1,)), ((), ())),
                                preferred_element_type=jnp.float32)
        a = ((g * jax.nn.sigmoid(g)) * u) * wcol
        out_ref[...] += jax.lax.dot_general(
            a, dwr[0], (((1,), (1,)), ((), ())),
            preferred_element_type=jnp.float32)

    @pl.when(e >= E)
    def _shared():
        g = jax.lax.dot_general(h, sgr[...], (((1,), (1,)), ((), ())),
                                preferred_element_type=jnp.float32)
        u = jax.lax.dot_general(h, sur[...], (((1,), (1,)), ((), ())),
                                preferred_element_type=jnp.float32)
        a = (g * jax.nn.sigmoid(g)) * u
        out_ref[...] += jax.lax.dot_general(
            a, sdr[...], (((1,), (1,)), ((), ())),
            preferred_element_type=jnp.float32)


def kernel(x, router_w, router_bias, gate_w, up_w, down_w,
           sh_gate_w, sh_up_w, sh_down_w):
    h = x.reshape(T, D)
    rb = router_bias.reshape(1, E)
    out = pl.pallas_call(
        _moe_body,
        grid=(E + 2, FJ),
        in_specs=[
            pl.BlockSpec((T, D), lambda e, j: (0, 0)),
            pl.BlockSpec((E, D), lambda e, j: (0, 0)),
            pl.BlockSpec((1, E), lambda e, j: (0, 0)),
            pl.BlockSpec((1, FB, D),
                         lambda e, j: (jnp.minimum(e, E - 1), j, 0)),
            pl.BlockSpec((1, FB, D),
                         lambda e, j: (jnp.minimum(e, E - 1), j, 0)),
            pl.BlockSpec((1, D, FB),
                         lambda e, j: (jnp.minimum(e, E - 1), 0, j)),
            pl.BlockSpec((SB, D),
                         lambda e, j: (jnp.where(e >= E, (e - E) * FJ + j, 0),
                                       0)),
            pl.BlockSpec((SB, D),
                         lambda e, j: (jnp.where(e >= E, (e - E) * FJ + j, 0),
                                       0)),
            pl.BlockSpec((D, SB),
                         lambda e, j: (0,
                                       jnp.where(e >= E, (e - E) * FJ + j,
                                                 0))),
        ],
        out_specs=pl.BlockSpec((T, D), lambda e, j: (0, 0)),
        out_shape=jax.ShapeDtypeStruct((T, D), jnp.float32),
        scratch_shapes=[pltpu.VMEM((T, E), jnp.float32)],
        compiler_params=pltpu.CompilerParams(
            dimension_semantics=("arbitrary", "arbitrary")),
    )(h, router_w, rb, gate_w, up_w, down_w, sh_gate_w, sh_up_w, sh_down_w)
    return out.reshape(x.shape)
